# baseline (speedup 1.0000x reference)
import functools

import jax
import jax.numpy as jnp
from jax import lax
from jax.experimental import pallas as pl
from jax.experimental.pallas import tpu as pltpu
from jax.experimental.pallas import tpu_sc as plsc

DIMH = 128
BATCH = 512
NC, NS, L = 2, 16, 16
NW = NC * NS
RPW = BATCH // NW
NCH = DIMH // L
NCLS = 13
NREL = 3
NOUT = 8


def _vsqrt(x):
    xc = jnp.maximum(x, jnp.float32(1e-30))
    i = lax.bitcast_convert_type(xc, jnp.int32)
    i = jnp.int32(0x5F3759DF) - jnp.right_shift(i, jnp.int32(1))
    g = lax.bitcast_convert_type(i, jnp.float32)
    for _ in range(3):
        g = g * (jnp.float32(1.5) - jnp.float32(0.5) * xc * g * g)
    return x * g


def _relu(x):
    return jnp.maximum(x, jnp.float32(0.0))


def _sc_body(idx_hbm, cls_hbm, rel_hbm, out_hbm, *refs):
    idx_v = refs[0]
    cbuf = refs[1:1 + NCLS]
    rbuf = refs[1 + NCLS:1 + NCLS + NREL]
    partials, sem = refs[1 + NCLS + NREL:]

    wid = lax.axis_index("s") * NC + lax.axis_index("c")
    base = wid * RPW

    for k in range(NCLS + NREL):
        pltpu.sync_copy(idx_hbm.at[pl.ds(k * BATCH + base, RPW)], idx_v.at[k])

    copies = []
    for k in range(NCLS):
        copies.append(pltpu.async_copy(cls_hbm.at[idx_v.at[k]], cbuf[k], sem))
    for k in range(NREL):
        copies.append(
            pltpu.async_copy(rel_hbm.at[idx_v.at[NCLS + k]], rbuf[k], sem))
    for cp in copies:
        cp.wait()

    zero = jnp.zeros((L,), jnp.float32)
    half = jnp.float32(0.5)
    iota = lax.iota(jnp.int32, L)

    def lo(ref, i, j):
        return ref[i, pl.ds(j * L, L)]

    def hi(ref, i, j):
        return jnp.abs(ref[i, pl.ds(DIMH + j * L, L)])

    def row_body(i, carry):
        s134, djr, negr, ar, br = carry
        dja = zero
        nega = zero
        aa = zero
        ba = zero
        for j in range(NCH):
            t = _relu(jnp.abs(lo(cbuf[0], i, j) - lo(cbuf[1], i, j))
                      + hi(cbuf[0], i, j) - hi(cbuf[1], i, j))
            s134 = s134 + t * t
            t = _relu(jnp.abs(lo(cbuf[5], i, j) + lo(rbuf[0], i, j)
                              - lo(cbuf[6], i, j))
                      + hi(cbuf[5], i, j) - hi(cbuf[6], i, j))
            s134 = s134 + t * t
            t = _relu(jnp.abs(lo(cbuf[7], i, j) - lo(rbuf[1], i, j)
                              - lo(cbuf[8], i, j))
                      - hi(cbuf[7], i, j) - hi(cbuf[8], i, j))
            s134 = s134 + t * t
            t = _relu(jnp.abs(lo(cbuf[9], i, j) - lo(cbuf[10], i, j))
                      - hi(cbuf[9], i, j) - hi(cbuf[10], i, j))
            dja = dja + t * t
            t = _relu(jnp.abs(lo(cbuf[11], i, j) + lo(rbuf[2], i, j)
                              - lo(cbuf[12], i, j))
                      - hi(cbuf[11], i, j) - hi(cbuf[12], i, j))
            nega = nega + t * t
            c1 = lo(cbuf[2], i, j)
            c2 = hi(cbuf[2], i, j)
            d1 = lo(cbuf[3], i, j)
            d2 = hi(cbuf[3], i, j)
            e1 = lo(cbuf[4], i, j)
            e2 = hi(cbuf[4], i, j)
            st = jnp.maximum(c1 - c2, d1 - d2)
            en = jnp.minimum(c1 + c2, d1 + d2)
            diff = st - en
            ta = _relu(jnp.abs(half * (st + en) - e1)
                       + half * jnp.abs(diff) - e2)
            aa = aa + ta * ta
            tb = _relu(diff)
            ba = ba + tb * tb
        m = iota == i
        zf = jnp.float32(0.0)
        djr = djr + jnp.where(m, jnp.sum(dja), zf)
        negr = negr + jnp.where(m, jnp.sum(nega), zf)
        ar = ar + jnp.where(m, jnp.sum(aa), zf)
        br = br + jnp.where(m, jnp.sum(ba), zf)
        return s134, djr, negr, ar, br

    s134, djr, negr, a2, b2 = lax.fori_loop(
        0, RPW, row_body, (zero, zero, zero, zero, zero))

    two = jnp.float32(2.0)
    djv = _relu(two - _vsqrt(djr))
    negv = two - _vsqrt(negr)

    partials[0, :] = s134
    partials[1, :] = a2
    partials[2, :] = _vsqrt(a2)
    partials[3, :] = b2
    partials[4, :] = _vsqrt(b2)
    partials[5, :] = djv * djv
    partials[6, :] = negv * negv
    partials[7, :] = zero
    pltpu.sync_copy(partials, out_hbm.at[wid])


def _finish_body(x_ref, o_ref):
    x = x_ref[...]
    inv = jnp.float32(1.0 / BATCH)
    s134 = jnp.sum(x[:, 0, :])
    sa2 = jnp.sum(x[:, 1, :])
    sa = jnp.sum(x[:, 2, :])
    sb2 = jnp.sum(x[:, 3, :])
    sb = jnp.sum(x[:, 4, :])
    sdj = jnp.sum(x[:, 5, :])
    sneg = jnp.sum(x[:, 6, :])
    loss2 = inv * sa2 + inv * sb2 + jnp.float32(2.0) * (inv * sa) * (inv * sb)
    total = inv * s134 + loss2 + inv * sdj + inv * sneg
    o_ref[...] = jnp.broadcast_to(total, (1, 1))


@jax.jit
def _run(idx_all, classEmb, relEmb):
    mesh = plsc.VectorSubcoreMesh(core_axis_name="c", subcore_axis_name="s")
    scratch = [pltpu.VMEM((L, RPW), jnp.int32)]
    scratch += [pltpu.VMEM((RPW, 2 * DIMH), jnp.float32) for _ in range(NCLS)]
    scratch += [pltpu.VMEM((RPW, DIMH), jnp.float32) for _ in range(NREL)]
    scratch += [pltpu.VMEM((NOUT, L), jnp.float32), pltpu.SemaphoreType.DMA]
    sc_call = pl.kernel(
        _sc_body,
        out_type=jax.ShapeDtypeStruct((NW, NOUT, L), jnp.float32),
        mesh=mesh,
        scratch_types=scratch,
        compiler_params=pltpu.CompilerParams(needs_layout_passes=False),
    )
    partials = sc_call(idx_all, classEmb, relEmb)
    out = pl.pallas_call(
        _finish_body,
        out_shape=jax.ShapeDtypeStruct((1, 1), jnp.float32),
    )(partials)
    return jnp.reshape(out, ())


def kernel(nf1, nf2, nf3, nf4, disjoint, nf3_neg, classEmb, relEmb):
    b = BATCH
    cols = [
        nf1[:b, 0], nf1[:b, 1],
        nf2[:b, 0], nf2[:b, 1], nf2[:b, 2],
        nf3[:b, 0], nf3[:b, 2],
        nf4[:b, 1], nf4[:b, 2],
        disjoint[:b, 0], disjoint[:b, 1],
        nf3_neg[:b, 0], nf3_neg[:b, 2],
        nf3[:b, 1], nf4[:b, 0], nf3_neg[:b, 1],
    ]
    idx_all = jnp.concatenate([c.astype(jnp.int32) for c in cols], axis=0)
    return _run(idx_all, classEmb, relEmb)


# speedup vs baseline: 1.2333x; 1.2333x over previous
import functools

import jax
import jax.numpy as jnp
from jax import lax
from jax.experimental import pallas as pl
from jax.experimental.pallas import tpu as pltpu
from jax.experimental.pallas import tpu_sc as plsc

DIMH = 128
BATCH = 512
NC, NS, L = 2, 16, 16
NW = NC * NS
RPW = BATCH // NW
NCH = DIMH // L
NCLS = 13
NREL = 3
NOUT = 8


def _vsqrt(x):
    xc = jnp.maximum(x, jnp.float32(1e-30))
    i = lax.bitcast_convert_type(xc, jnp.int32)
    i = jnp.int32(0x5F3759DF) - jnp.right_shift(i, jnp.int32(1))
    g = lax.bitcast_convert_type(i, jnp.float32)
    for _ in range(3):
        g = g * (jnp.float32(1.5) - jnp.float32(0.5) * xc * g * g)
    return x * g


def _relu(x):
    return jnp.maximum(x, jnp.float32(0.0))


def _sc_body(idx_hbm, cls_hbm, rel_hbm, out_hbm, *refs):
    idx_v, bufa, bufb, bufr, partials, sem = refs

    wid = lax.axis_index("s") * NC + lax.axis_index("c")

    nidx = (NCLS + NREL) * RPW
    pltpu.sync_copy(idx_hbm.at[pl.ds(wid * nidx, nidx)], idx_v)

    c1 = pltpu.async_copy(cls_hbm.at[idx_v.at[pl.ds(0, 7 * RPW)]], bufa, sem)
    c2 = pltpu.async_copy(
        cls_hbm.at[idx_v.at[pl.ds(7 * RPW, 6 * RPW)]], bufb, sem)
    c3 = pltpu.async_copy(
        rel_hbm.at[idx_v.at[pl.ds(NCLS * RPW, NREL * RPW)]], bufr, sem)
    c1.wait()
    c2.wait()
    c3.wait()

    zero = jnp.zeros((L,), jnp.float32)
    half = jnp.float32(0.5)
    iota = lax.iota(jnp.int32, L)

    def _cref(k):
        return (bufa, k * RPW) if k < 7 else (bufb, (k - 7) * RPW)

    def lo(k, i, j):
        ref, off = _cref(k)
        return ref[off + i, pl.ds(j * L, L)]

    def hi(k, i, j):
        ref, off = _cref(k)
        return jnp.abs(ref[off + i, pl.ds(DIMH + j * L, L)])

    def rlo(k, i, j):
        return bufr[k * RPW + i, pl.ds(j * L, L)]

    def row_body(i, carry):
        s134, djr, negr, ar, br = carry
        dja = zero
        nega = zero
        aa = zero
        ba = zero
        for j in range(NCH):
            t = _relu(jnp.abs(lo(0, i, j) - lo(1, i, j))
                      + hi(0, i, j) - hi(1, i, j))
            s134 = s134 + t * t
            t = _relu(jnp.abs(lo(5, i, j) + rlo(0, i, j) - lo(6, i, j))
                      + hi(5, i, j) - hi(6, i, j))
            s134 = s134 + t * t
            t = _relu(jnp.abs(lo(7, i, j) - rlo(1, i, j) - lo(8, i, j))
                      - hi(7, i, j) - hi(8, i, j))
            s134 = s134 + t * t
            t = _relu(jnp.abs(lo(9, i, j) - lo(10, i, j))
                      - hi(9, i, j) - hi(10, i, j))
            dja = dja + t * t
            t = _relu(jnp.abs(lo(11, i, j) + rlo(2, i, j) - lo(12, i, j))
                      - hi(11, i, j) - hi(12, i, j))
            nega = nega + t * t
            c1 = lo(2, i, j)
            c2 = hi(2, i, j)
            d1 = lo(3, i, j)
            d2 = hi(3, i, j)
            e1 = lo(4, i, j)
            e2 = hi(4, i, j)
            st = jnp.maximum(c1 - c2, d1 - d2)
            en = jnp.minimum(c1 + c2, d1 + d2)
            diff = st - en
            ta = _relu(jnp.abs(half * (st + en) - e1)
                       + half * jnp.abs(diff) - e2)
            aa = aa + ta * ta
            tb = _relu(diff)
            ba = ba + tb * tb
        m = iota == i
        zf = jnp.float32(0.0)
        djr = djr + jnp.where(m, jnp.sum(dja), zf)
        negr = negr + jnp.where(m, jnp.sum(nega), zf)
        ar = ar + jnp.where(m, jnp.sum(aa), zf)
        br = br + jnp.where(m, jnp.sum(ba), zf)
        return s134, djr, negr, ar, br

    s134, djr, negr, a2, b2 = lax.fori_loop(
        0, RPW, row_body, (zero, zero, zero, zero, zero))

    two = jnp.float32(2.0)
    djv = _relu(two - _vsqrt(djr))
    negv = two - _vsqrt(negr)

    partials[0, :] = s134
    partials[1, :] = a2
    partials[2, :] = _vsqrt(a2)
    partials[3, :] = b2
    partials[4, :] = _vsqrt(b2)
    partials[5, :] = djv * djv
    partials[6, :] = negv * negv
    partials[7, :] = zero
    pltpu.sync_copy(partials, out_hbm.at[wid])


def _finish_body(x_ref, o_ref):
    x = x_ref[...]
    inv = jnp.float32(1.0 / BATCH)
    s134 = jnp.sum(x[:, 0, :])
    sa2 = jnp.sum(x[:, 1, :])
    sa = jnp.sum(x[:, 2, :])
    sb2 = jnp.sum(x[:, 3, :])
    sb = jnp.sum(x[:, 4, :])
    sdj = jnp.sum(x[:, 5, :])
    sneg = jnp.sum(x[:, 6, :])
    loss2 = inv * sa2 + inv * sb2 + jnp.float32(2.0) * (inv * sa) * (inv * sb)
    total = inv * s134 + loss2 + inv * sdj + inv * sneg
    o_ref[...] = jnp.broadcast_to(total, (1, 1))


@jax.jit
def _run(idx_all, classEmb, relEmb):
    mesh = plsc.VectorSubcoreMesh(core_axis_name="c", subcore_axis_name="s")
    scratch = [
        pltpu.VMEM(((NCLS + NREL) * RPW,), jnp.int32),
        pltpu.VMEM((7 * RPW, 2 * DIMH), jnp.float32),
        pltpu.VMEM((6 * RPW, 2 * DIMH), jnp.float32),
        pltpu.VMEM((NREL * RPW, DIMH), jnp.float32),
        pltpu.VMEM((NOUT, L), jnp.float32),
        pltpu.SemaphoreType.DMA,
    ]
    sc_call = pl.kernel(
        _sc_body,
        out_type=jax.ShapeDtypeStruct((NW, NOUT, L), jnp.float32),
        mesh=mesh,
        scratch_types=scratch,
        compiler_params=pltpu.CompilerParams(needs_layout_passes=False),
    )
    partials = sc_call(idx_all, classEmb, relEmb)
    out = pl.pallas_call(
        _finish_body,
        out_shape=jax.ShapeDtypeStruct((1, 1), jnp.float32),
    )(partials)
    return jnp.reshape(out, ())


def kernel(nf1, nf2, nf3, nf4, disjoint, nf3_neg, classEmb, relEmb):
    b = BATCH
    cols = [
        nf1[:b, 0], nf1[:b, 1],
        nf2[:b, 0], nf2[:b, 1], nf2[:b, 2],
        nf3[:b, 0], nf3[:b, 2],
        nf4[:b, 1], nf4[:b, 2],
        disjoint[:b, 0], disjoint[:b, 1],
        nf3_neg[:b, 0], nf3_neg[:b, 2],
        nf3[:b, 1], nf4[:b, 0], nf3_neg[:b, 1],
    ]
    idx_all = jnp.stack([c.astype(jnp.int32) for c in cols], axis=0)
    idx3 = idx_all.reshape(16, NW, RPW).transpose(1, 0, 2).reshape(NW * 256)
    return _run(idx3, classEmb, relEmb)
